# baseline (speedup 1.0000x reference)
import jax
import jax.numpy as jnp
from jax.experimental import pallas as pl
from jax.experimental.pallas import tpu as pltpu

_B, _L, _D = 512, 512, 256
_G = 16


def _body(mask_sref, y_hbm, z_hbm, st_ref, mk_ref, yi_blk, zi_blk,
          yo_hbm, zo_hbm, so_ref, sems):
    b = pl.program_id(0)
    n = pl.num_programs(0)

    @pl.when(b == 0)
    def _():
        so_ref[...] = jnp.where(mk_ref[...] != 0,
                                jnp.zeros_like(st_ref[...]), st_ref[...])

    def _wait_step(step):
        base = step * _G
        for j in range(_G):
            pltpu.make_async_copy(y_hbm.at[pl.ds(base + j, 1)],
                                  yo_hbm.at[pl.ds(base + j, 1)],
                                  sems.at[0, j]).wait()
            pltpu.make_async_copy(z_hbm.at[pl.ds(base + j, 1)],
                                  zo_hbm.at[pl.ds(base + j, 1)],
                                  sems.at[1, j]).wait()

    @pl.when(b > 0)
    def _():
        _wait_step(b - 1)

    base = b * _G
    for j in range(_G):
        @pl.when(mask_sref[base + j] == 0)
        def _(j=j):
            pltpu.make_async_copy(y_hbm.at[pl.ds(base + j, 1)],
                                  yo_hbm.at[pl.ds(base + j, 1)],
                                  sems.at[0, j]).start()
            pltpu.make_async_copy(z_hbm.at[pl.ds(base + j, 1)],
                                  zo_hbm.at[pl.ds(base + j, 1)],
                                  sems.at[1, j]).start()

        @pl.when(mask_sref[base + j] != 0)
        def _(j=j):
            pltpu.make_async_copy(yi_blk,
                                  yo_hbm.at[pl.ds(base + j, 1)],
                                  sems.at[0, j]).start()
            pltpu.make_async_copy(zi_blk,
                                  zo_hbm.at[pl.ds(base + j, 1)],
                                  sems.at[1, j]).start()

    @pl.when(b == n - 1)
    def _():
        _wait_step(b)


def kernel(y, z, steps, mask, y_init, z_init):
    B, L, D = y.shape
    mask_i32 = mask.astype(jnp.int32)

    steps2d = steps.reshape(1, B)
    mask2d = mask_i32.reshape(1, B)
    yi_row = jnp.broadcast_to(y_init, (1, L, D))
    zi_row = jnp.broadcast_to(z_init, (1, L, D))

    grid_spec = pltpu.PrefetchScalarGridSpec(
        num_scalar_prefetch=1,
        grid=(B // _G,),
        in_specs=[
            pl.BlockSpec(memory_space=pltpu.MemorySpace.HBM),
            pl.BlockSpec(memory_space=pltpu.MemorySpace.HBM),
            pl.BlockSpec((1, B), lambda i, mref: (0, 0)),
            pl.BlockSpec((1, B), lambda i, mref: (0, 0)),
            pl.BlockSpec((1, L, D), lambda i, mref: (0, 0, 0)),
            pl.BlockSpec((1, L, D), lambda i, mref: (0, 0, 0)),
        ],
        out_specs=[
            pl.BlockSpec(memory_space=pltpu.MemorySpace.HBM),
            pl.BlockSpec(memory_space=pltpu.MemorySpace.HBM),
            pl.BlockSpec((1, B), lambda i, mref: (0, 0)),
        ],
        scratch_shapes=[
            pltpu.SemaphoreType.DMA((2, _G)),
        ],
    )

    y_new, z_new, so = pl.pallas_call(
        _body,
        grid_spec=grid_spec,
        out_shape=[
            jax.ShapeDtypeStruct((B, L, D), y.dtype),
            jax.ShapeDtypeStruct((B, L, D), z.dtype),
            jax.ShapeDtypeStruct((1, B), steps.dtype),
        ],
        compiler_params=pltpu.CompilerParams(
            dimension_semantics=("arbitrary",),
        ),
    )(mask_i32, y, z, steps2d, mask2d, yi_row, zi_row)

    return (y_new, z_new, so.reshape(B))


# speedup vs baseline: 26.7073x; 26.7073x over previous
import jax
import jax.numpy as jnp
from jax import lax
from jax.experimental import pallas as pl
from jax.experimental.pallas import tpu as pltpu
from jax.experimental.pallas import tpu_sc as plsc

_B, _L, _D = 512, 512, 256
_G = 16
_NC, _NS = 2, 16
_NW = _NC * _NS
_RPW = _B // _NW
_CH = 128
_NCH = _L // _CH


def _tc_body(mask_sref, y_hbm, st_ref, mk_ref, yi_ref, yo_ref, so_ref, sems):
    b = pl.program_id(0)
    base = b * _G

    @pl.when(b == 0)
    def _():
        so_ref[...] = jnp.where(mk_ref[...] != 0,
                                jnp.zeros_like(st_ref[...]), st_ref[...])

    for j in range(_G):
        @pl.when(mask_sref[base + j] == 0)
        def _(j=j):
            pltpu.make_async_copy(y_hbm.at[pl.ds(base + j, 1)],
                                  yo_ref.at[pl.ds(j, 1)], sems.at[j]).start()

    yi_row = jnp.broadcast_to(yi_ref[...].reshape(1, 1, _D), (1, _L, _D))
    for j in range(_G):
        @pl.when(mask_sref[base + j] != 0)
        def _(j=j):
            yo_ref[pl.ds(j, 1)] = yi_row

    for j in range(_G):
        @pl.when(mask_sref[base + j] == 0)
        def _(j=j):
            pltpu.make_async_copy(y_hbm.at[pl.ds(base + j, 1)],
                                  yo_ref.at[pl.ds(j, 1)], sems.at[j]).wait()


def _tc_call(y, steps, mask_i32, y_init):
    B, L, D = y.shape
    steps2d = steps.reshape(1, B)
    mask2d = mask_i32.reshape(1, B)
    yi2d = y_init.reshape(1, D)

    grid_spec = pltpu.PrefetchScalarGridSpec(
        num_scalar_prefetch=1,
        grid=(B // _G,),
        in_specs=[
            pl.BlockSpec(memory_space=pltpu.MemorySpace.HBM),
            pl.BlockSpec((1, B), lambda i, mref: (0, 0)),
            pl.BlockSpec((1, B), lambda i, mref: (0, 0)),
            pl.BlockSpec((1, D), lambda i, mref: (0, 0)),
        ],
        out_specs=[
            pl.BlockSpec((_G, L, D), lambda i, mref: (i, 0, 0)),
            pl.BlockSpec((1, B), lambda i, mref: (0, 0)),
        ],
        scratch_shapes=[
            pltpu.SemaphoreType.DMA((_G,)),
        ],
    )

    y_new, so = pl.pallas_call(
        _tc_body,
        grid_spec=grid_spec,
        out_shape=[
            jax.ShapeDtypeStruct((B, L, D), y.dtype),
            jax.ShapeDtypeStruct((1, B), steps.dtype),
        ],
        compiler_params=pltpu.CompilerParams(
            dimension_semantics=("arbitrary",),
        ),
    )(mask_i32, y, steps2d, mask2d, yi2d)
    return y_new, so.reshape(B)


def _sc_body(z_hbm, mask_hbm, zi_hbm, zo_hbm, mvec, zinit_v, buf0, buf1,
             sem_init, sem_w0, sem_w1):
    wid = lax.axis_index("s") * _NC + lax.axis_index("c")
    base = wid * _RPW

    pltpu.sync_copy(mask_hbm.at[pl.ds(base, _RPW)], mvec)
    pltpu.sync_copy(zi_hbm, zinit_v)
    m = mvec[...]

    for j in range(_RPW):
        row = base + j
        s = m[j]

        @pl.when(s != 0)
        def _(row=row):
            for k in range(_NCH):
                pltpu.async_copy(
                    zinit_v, zo_hbm.at[pl.ds(row, 1), pl.ds(k * _CH, _CH)],
                    sem_init)
            for k in range(_NCH):
                pltpu.make_async_copy(
                    zinit_v, zo_hbm.at[pl.ds(row, 1), pl.ds(k * _CH, _CH)],
                    sem_init).wait()

        @pl.when(s == 0)
        def _(row=row):
            for k in range(_NCH):
                buf = buf0 if k % 2 == 0 else buf1
                sem = sem_w0 if k % 2 == 0 else sem_w1
                if k >= 2:
                    pltpu.make_async_copy(
                        buf, zo_hbm.at[pl.ds(row, 1), pl.ds((k - 2) * _CH, _CH)],
                        sem).wait()
                pltpu.sync_copy(
                    z_hbm.at[pl.ds(row, 1), pl.ds(k * _CH, _CH)], buf)
                pltpu.async_copy(
                    buf, zo_hbm.at[pl.ds(row, 1), pl.ds(k * _CH, _CH)], sem)
            for k in range(_NCH - 2, _NCH):
                buf = buf0 if k % 2 == 0 else buf1
                sem = sem_w0 if k % 2 == 0 else sem_w1
                pltpu.make_async_copy(
                    buf, zo_hbm.at[pl.ds(row, 1), pl.ds(k * _CH, _CH)],
                    sem).wait()


def _sc_call(z, mask_i32, z_init):
    B, L, D = z.shape
    zi_chunk = jnp.broadcast_to(z_init, (1, _CH, D))

    import functools
    kfn = functools.partial(
        pl.kernel,
        mesh=plsc.VectorSubcoreMesh(core_axis_name="c", subcore_axis_name="s"),
        out_type=jax.ShapeDtypeStruct((B, L, D), z.dtype),
        scratch_types=[
            pltpu.VMEM((_RPW,), jnp.int32),
            pltpu.VMEM((1, _CH, D), jnp.float32),
            pltpu.VMEM((1, _CH, D), jnp.float32),
            pltpu.VMEM((1, _CH, D), jnp.float32),
            pltpu.SemaphoreType.DMA,
            pltpu.SemaphoreType.DMA,
            pltpu.SemaphoreType.DMA,
        ],
    )(_sc_body)
    return kfn(z, mask_i32, zi_chunk)


def kernel(y, z, steps, mask, y_init, z_init):
    mask_i32 = mask.astype(jnp.int32)
    y_new, steps_new = _tc_call(y, steps, mask_i32, y_init)
    z_new = _sc_call(z, mask_i32, z_init)
    return (y_new, z_new, steps_new)


# speedup vs baseline: 26.8053x; 1.0037x over previous
import jax
import jax.numpy as jnp
from jax import lax
from jax.experimental import pallas as pl
from jax.experimental.pallas import tpu as pltpu
from jax.experimental.pallas import tpu_sc as plsc

_B, _L, _D = 512, 512, 256
_G = 16
_NC, _NS = 2, 16
_NW = _NC * _NS
_RPW = _B // _NW
_CH = 128
_NCH = _L // _CH


def _tc_body(mask_sref, y_hbm, st_ref, mk_ref, yi_ref, yo_ref, so_ref, sems):
    b = pl.program_id(0)
    base = b * _G

    @pl.when(b == 0)
    def _():
        so_ref[...] = jnp.where(mk_ref[...] != 0,
                                jnp.zeros_like(st_ref[...]), st_ref[...])

    for j in range(_G):
        @pl.when(mask_sref[base + j] == 0)
        def _(j=j):
            pltpu.make_async_copy(y_hbm.at[pl.ds(base + j, 1)],
                                  yo_ref.at[pl.ds(j, 1)], sems.at[j]).start()

    yi_row = jnp.broadcast_to(yi_ref[...].reshape(1, 1, _D), (1, _L, _D))
    for j in range(_G):
        @pl.when(mask_sref[base + j] != 0)
        def _(j=j):
            yo_ref[pl.ds(j, 1)] = yi_row

    for j in range(_G):
        @pl.when(mask_sref[base + j] == 0)
        def _(j=j):
            pltpu.make_async_copy(y_hbm.at[pl.ds(base + j, 1)],
                                  yo_ref.at[pl.ds(j, 1)], sems.at[j]).wait()


def _tc_call(y, steps, mask_i32, y_init):
    B, L, D = y.shape
    steps2d = steps.reshape(1, B)
    mask2d = mask_i32.reshape(1, B)
    yi2d = y_init.reshape(1, D)

    grid_spec = pltpu.PrefetchScalarGridSpec(
        num_scalar_prefetch=1,
        grid=(B // _G,),
        in_specs=[
            pl.BlockSpec(memory_space=pltpu.MemorySpace.HBM),
            pl.BlockSpec((1, B), lambda i, mref: (0, 0)),
            pl.BlockSpec((1, B), lambda i, mref: (0, 0)),
            pl.BlockSpec((1, D), lambda i, mref: (0, 0)),
        ],
        out_specs=[
            pl.BlockSpec((_G, L, D), lambda i, mref: (i, 0, 0)),
            pl.BlockSpec((1, B), lambda i, mref: (0, 0)),
        ],
        scratch_shapes=[
            pltpu.SemaphoreType.DMA((_G,)),
        ],
    )

    y_new, so = pl.pallas_call(
        _tc_body,
        grid_spec=grid_spec,
        out_shape=[
            jax.ShapeDtypeStruct((B, L, D), y.dtype),
            jax.ShapeDtypeStruct((1, B), steps.dtype),
        ],
        compiler_params=pltpu.CompilerParams(
            dimension_semantics=("arbitrary",),
        ),
    )(mask_i32, y, steps2d, mask2d, yi2d)
    return y_new, so.reshape(B)


def _sc_body(z_hbm, mask_hbm, zi_hbm, zo_hbm, mvec, zinit_v, buf0, buf1,
             sem_r0, sem_r1, sem_w0, sem_w1):
    wid = lax.axis_index("s") * _NC + lax.axis_index("c")
    base = wid * _RPW

    pltpu.sync_copy(mask_hbm.at[pl.ds(base, _RPW)], mvec)
    pltpu.sync_copy(zi_hbm, zinit_v)
    m = mvec[...]
    bufs = (buf0, buf1)
    sem_r = (sem_r0, sem_r1)
    sem_w = (sem_w0, sem_w1)

    t = 0
    for j in range(_RPW):
        row = base + j
        s = m[j]
        for k in range(_NCH):
            slot = t % 2
            dst = zo_hbm.at[pl.ds(row, 1), pl.ds(k * _CH, _CH)]
            if t >= 2:
                pltpu.make_async_copy(zinit_v, dst, sem_w[slot]).wait()

            @pl.when(s == 0)
            def _(dst=dst, slot=slot, row=row, k=k):
                src_slice = z_hbm.at[pl.ds(row, 1), pl.ds(k * _CH, _CH)]
                pltpu.async_copy(src_slice, bufs[slot], sem_r[slot])
                pltpu.make_async_copy(src_slice, bufs[slot],
                                      sem_r[slot]).wait()
                pltpu.async_copy(bufs[slot], dst, sem_w[slot])

            @pl.when(s != 0)
            def _(dst=dst, slot=slot):
                pltpu.async_copy(zinit_v, dst, sem_w[slot])

            t += 1

    for slot in (0, 1):
        pltpu.make_async_copy(
            zinit_v, zo_hbm.at[pl.ds(base, 1), pl.ds(0, _CH)],
            sem_w[slot]).wait()


def _sc_call(z, mask_i32, z_init):
    B, L, D = z.shape
    zi_chunk = jnp.broadcast_to(z_init, (1, _CH, D))

    import functools
    kfn = functools.partial(
        pl.kernel,
        mesh=plsc.VectorSubcoreMesh(core_axis_name="c", subcore_axis_name="s"),
        out_type=jax.ShapeDtypeStruct((B, L, D), z.dtype),
        scratch_types=[
            pltpu.VMEM((_RPW,), jnp.int32),
            pltpu.VMEM((1, _CH, D), jnp.float32),
            pltpu.VMEM((1, _CH, D), jnp.float32),
            pltpu.VMEM((1, _CH, D), jnp.float32),
            pltpu.SemaphoreType.DMA,
            pltpu.SemaphoreType.DMA,
            pltpu.SemaphoreType.DMA,
            pltpu.SemaphoreType.DMA,
        ],
    )(_sc_body)
    return kfn(z, mask_i32, zi_chunk)


def kernel(y, z, steps, mask, y_init, z_init):
    mask_i32 = mask.astype(jnp.int32)
    y_new, steps_new = _tc_call(y, steps, mask_i32, y_init)
    z_new = _sc_call(z, mask_i32, z_init)
    return (y_new, z_new, steps_new)


# speedup vs baseline: 27.9464x; 1.0426x over previous
import jax
import jax.numpy as jnp
from jax import lax
from jax.experimental import pallas as pl
from jax.experimental.pallas import tpu as pltpu
from jax.experimental.pallas import tpu_sc as plsc

_B, _L, _D = 512, 512, 256
_G = 16
_NC, _NS = 2, 16
_NW = _NC * _NS
_RPW = _B // _NW
_CH = 128
_NCH = _L // _CH


def _tc_body(mask_sref, y_hbm, st_ref, mk_ref, yi_ref, yo_ref, so_ref, sems):
    b = pl.program_id(0)
    base = b * _G

    @pl.when(b == 0)
    def _():
        so_ref[...] = jnp.where(mk_ref[...] != 0,
                                jnp.zeros_like(st_ref[...]), st_ref[...])

    for j in range(_G):
        @pl.when(mask_sref[base + j] == 0)
        def _(j=j):
            pltpu.make_async_copy(y_hbm.at[pl.ds(base + j, 1)],
                                  yo_ref.at[pl.ds(j, 1)], sems.at[j]).start()

    yi_row = jnp.broadcast_to(yi_ref[...].reshape(1, 1, _D), (1, _L, _D))
    for j in range(_G):
        @pl.when(mask_sref[base + j] != 0)
        def _(j=j):
            yo_ref[pl.ds(j, 1)] = yi_row

    for j in range(_G):
        @pl.when(mask_sref[base + j] == 0)
        def _(j=j):
            pltpu.make_async_copy(y_hbm.at[pl.ds(base + j, 1)],
                                  yo_ref.at[pl.ds(j, 1)], sems.at[j]).wait()


def _tc_call(y, steps, mask_i32, y_init):
    B, L, D = y.shape
    steps2d = steps.reshape(1, B)
    mask2d = mask_i32.reshape(1, B)
    yi2d = y_init.reshape(1, D)

    grid_spec = pltpu.PrefetchScalarGridSpec(
        num_scalar_prefetch=1,
        grid=(B // _G,),
        in_specs=[
            pl.BlockSpec(memory_space=pltpu.MemorySpace.HBM),
            pl.BlockSpec((1, B), lambda i, mref: (0, 0)),
            pl.BlockSpec((1, B), lambda i, mref: (0, 0)),
            pl.BlockSpec((1, D), lambda i, mref: (0, 0)),
        ],
        out_specs=[
            pl.BlockSpec((_G, L, D), lambda i, mref: (i, 0, 0)),
            pl.BlockSpec((1, B), lambda i, mref: (0, 0)),
        ],
        scratch_shapes=[
            pltpu.SemaphoreType.DMA((_G,)),
        ],
    )

    y_new, so = pl.pallas_call(
        _tc_body,
        grid_spec=grid_spec,
        out_shape=[
            jax.ShapeDtypeStruct((B, L, D), y.dtype),
            jax.ShapeDtypeStruct((1, B), steps.dtype),
        ],
        compiler_params=pltpu.CompilerParams(
            dimension_semantics=("arbitrary",),
        ),
    )(mask_i32, y, steps2d, mask2d, yi2d)
    return y_new, so.reshape(B)


def _sc_body(z_hbm, mask_hbm, zi_hbm, zo_hbm, mvec, shbuf, shinit,
             sem_r0, sem_r1, sem_w0, sem_w1):
    cid = lax.axis_index("c")
    sid = lax.axis_index("s")
    wid = sid * _NC + cid
    base = wid * _RPW

    pltpu.sync_copy(mask_hbm.at[pl.ds(base, _RPW)], mvec)
    pltpu.sync_copy(zi_hbm, shinit.at[sid])
    m = mvec[...]
    sem_r = (sem_r0, sem_r1)
    sem_w = (sem_w0, sem_w1)

    t = 0
    for j in range(_RPW):
        row = base + j
        s = m[j]
        for k in range(_NCH):
            slot = t % 2
            dst = zo_hbm.at[pl.ds(row, 1), pl.ds(k * _CH, _CH)]
            buf = shbuf.at[sid, slot]
            if t >= 2:
                pltpu.make_async_copy(shinit.at[sid], dst, sem_w[slot]).wait()

            @pl.when(s == 0)
            def _(dst=dst, slot=slot, row=row, k=k, buf=buf):
                src_slice = z_hbm.at[pl.ds(row, 1), pl.ds(k * _CH, _CH)]
                pltpu.async_copy(src_slice, buf, sem_r[slot])
                pltpu.make_async_copy(src_slice, buf, sem_r[slot]).wait()
                pltpu.async_copy(buf, dst, sem_w[slot])

            @pl.when(s != 0)
            def _(dst=dst, slot=slot):
                pltpu.async_copy(shinit.at[sid], dst, sem_w[slot])

            t += 1

    for slot in (0, 1):
        pltpu.make_async_copy(
            shinit.at[sid], zo_hbm.at[pl.ds(base, 1), pl.ds(0, _CH)],
            sem_w[slot]).wait()


def _sc_call(z, mask_i32, z_init):
    B, L, D = z.shape
    zi_chunk = jnp.broadcast_to(z_init, (1, _CH, D))

    import functools
    kfn = functools.partial(
        pl.kernel,
        mesh=plsc.VectorSubcoreMesh(core_axis_name="c", subcore_axis_name="s"),
        out_type=jax.ShapeDtypeStruct((B, L, D), z.dtype),
        scratch_types=[
            pltpu.VMEM((_RPW,), jnp.int32),
            pltpu.VMEM_SHARED((_NS, 2, 1, _CH, _D), jnp.float32),
            pltpu.VMEM_SHARED((_NS, 1, _CH, _D), jnp.float32),
            pltpu.SemaphoreType.DMA,
            pltpu.SemaphoreType.DMA,
            pltpu.SemaphoreType.DMA,
            pltpu.SemaphoreType.DMA,
        ],
    )(_sc_body)
    return kfn(z, mask_i32, zi_chunk)


def kernel(y, z, steps, mask, y_init, z_init):
    mask_i32 = mask.astype(jnp.int32)
    y_new, steps_new = _tc_call(y, steps, mask_i32, y_init)
    z_new = _sc_call(z, mask_i32, z_init)
    return (y_new, z_new, steps_new)
